# baseline (speedup 1.0000x reference)
import jax
import jax.numpy as jnp
from jax.experimental import pallas as pl
from jax.experimental.pallas import tpu as pltpu

_B = 64
_BB = 4
_NBLK = _B // _BB


def _berhu_body(x_ref, y_ref, out_ref, acc_ref, vacc_ref, cache_ref):
    p = pl.program_id(0)
    j = pl.program_id(1)

    @pl.when(p == 0)
    def _():
        d = jnp.abs(x_ref[...] - y_ref[...])

        @pl.when(j == 0)
        def _():
            acc_ref[0] = 0.0
            acc_ref[1] = 0.0

        acc_ref[0] = jnp.maximum(acc_ref[0], jnp.max(d))
        acc_ref[1] += jnp.sum(d)
        cache_ref[pl.ds(j * _BB, _BB)] = d.astype(jnp.bfloat16)

    @pl.when(p == 1)
    def _():
        @pl.when(j == 0)
        def _():
            vacc_ref[...] = jnp.zeros_like(vacc_ref)

        cb = jnp.maximum(acc_ref[0] * 0.2, 0.0001).astype(jnp.bfloat16)
        d = cache_ref[pl.ds(j * _BB, _BB)].reshape(_BB * 512, 512)
        u = jnp.maximum(d - cb, jnp.bfloat16(0.0))
        ones = jnp.ones((_BB * 512,), jnp.bfloat16)
        sq = jax.lax.dot_general(ones, u * u, (((0,), (0,)), ((), ())),
                                 preferred_element_type=jnp.float32)
        vacc_ref[0, :] += sq

        @pl.when(j == _NBLK - 1)
        def _():
            c32 = cb.astype(jnp.float32)
            out_ref[0] = acc_ref[1] + jnp.sum(vacc_ref[0, :]) / (2.0 * c32)


def kernel(prediction, target):
    spec = pl.BlockSpec(
        (_BB, 1, 512, 512), lambda p, j: (jnp.where(p == 0, j, 0), 0, 0, 0))
    out = pl.pallas_call(
        _berhu_body,
        grid=(2, _NBLK),
        in_specs=[spec, spec],
        out_specs=pl.BlockSpec(memory_space=pltpu.SMEM),
        out_shape=jax.ShapeDtypeStruct((1,), jnp.float32),
        scratch_shapes=[
            pltpu.SMEM((4,), jnp.float32),
            pltpu.VMEM((1, 512), jnp.float32),
            pltpu.VMEM((_B, 1, 512, 512), jnp.bfloat16),
        ],
        compiler_params=pltpu.CompilerParams(
            dimension_semantics=("arbitrary", "arbitrary"),
            vmem_limit_bytes=64 * 1024 * 1024,
        ),
    )(prediction, target)
    return out.reshape(())


import functools
from jax.experimental.pallas import tpu_sc as plsc

_ROWS = _B * 512
_TAIL0 = 28672 // 32


def _sc_tail_max(xf, yf):
    mesh = plsc.VectorSubcoreMesh(core_axis_name="c", subcore_axis_name="s")

    @functools.partial(
        pl.kernel,
        out_type=jax.ShapeDtypeStruct((32, 16), jnp.float32),
        mesh=mesh,
        scratch_types=[pltpu.VMEM((16,), jnp.float32)],
    )
    def k(x_hbm, y_hbm, o_hbm, acc):
        wid = jax.lax.axis_index("s") * 2 + jax.lax.axis_index("c")
        acc[...] = jnp.zeros((16,), jnp.float32)

        def body(x_vmem, y_vmem):
            @pl.loop(0, 32)
            def _(r):
                @pl.loop(0, 512 // 16)
                def _(k16):
                    xv = x_vmem[r, pl.ds(k16 * 16, 16)]
                    yv = y_vmem[r, pl.ds(k16 * 16, 16)]
                    acc[...] = jnp.maximum(acc[...], jnp.abs(xv - yv))

        pltpu.emit_pipeline(
            body,
            grid=(2, 64),
            in_specs=[
                pl.BlockSpec((32, 512),
                             index_map=lambda i, j: (_TAIL0 + i * 64 + j, 0)),
                pl.BlockSpec((32, 512),
                             index_map=lambda i, j: (_TAIL0 + i * 64 + j, 0)),
            ],
            core_axis_name=("c", "s"),
            dimension_semantics=(pltpu.PARALLEL, pltpu.PARALLEL),
        )(x_hbm, y_hbm)
        pltpu.sync_copy(acc, o_hbm.at[wid])

    return k(xf, yf)


_tc_kernel = kernel


def kernel(prediction, target):
    loss = _tc_kernel(prediction, target)
    xf = prediction.reshape(_ROWS, 512)
    yf = target.reshape(_ROWS, 512)
    sc_max = jnp.max(_sc_tail_max(xf, yf))
    return loss + jnp.where(sc_max >= 0.0, 0.0, sc_max)


# speedup vs baseline: 1.5162x; 1.5162x over previous
import jax
import jax.numpy as jnp
from jax.experimental import pallas as pl
from jax.experimental.pallas import tpu as pltpu

_B = 64
_BB = 4
_NBLK = _B // _BB


def _berhu_body(x_ref, y_ref, out_ref, acc_ref, vacc_ref, cache_ref):
    p = pl.program_id(0)
    j = pl.program_id(1)

    @pl.when(p == 0)
    def _():
        d = jnp.abs(x_ref[...] - y_ref[...])

        @pl.when(j == 0)
        def _():
            acc_ref[0] = 0.0
            acc_ref[1] = 0.0

        acc_ref[0] = jnp.maximum(acc_ref[0], jnp.max(d))
        acc_ref[1] += jnp.sum(d)
        cache_ref[pl.ds(j * _BB, _BB)] = d.astype(jnp.bfloat16)

    @pl.when(jnp.logical_and(p == 1, j < _NBLK // 2))
    def _():
        @pl.when(j == 0)
        def _():
            vacc_ref[...] = jnp.zeros_like(vacc_ref)

        cb = jnp.maximum(acc_ref[0] * 0.2, 0.0001).astype(jnp.bfloat16)
        d = cache_ref[pl.ds(j * 2 * _BB, 2 * _BB)].reshape(2 * _BB * 512, 512)
        u = jnp.maximum(d - cb, jnp.bfloat16(0.0))
        ones = jnp.ones((2 * _BB * 512,), jnp.bfloat16)
        sq = jax.lax.dot_general(ones, u * u, (((0,), (0,)), ((), ())),
                                 preferred_element_type=jnp.float32)
        vacc_ref[0, :] += sq

    @pl.when(jnp.logical_and(p == 1, j == _NBLK - 1))
    def _():
        cb = jnp.maximum(acc_ref[0] * 0.2, 0.0001).astype(jnp.bfloat16)
        c32 = cb.astype(jnp.float32)
        out_ref[0] = acc_ref[1] + jnp.sum(vacc_ref[0, :]) / (2.0 * c32)


def kernel(prediction, target):
    spec = pl.BlockSpec(
        (_BB, 1, 512, 512), lambda p, j: (jnp.where(p == 0, j, 0), 0, 0, 0))
    out = pl.pallas_call(
        _berhu_body,
        grid=(2, _NBLK),
        in_specs=[spec, spec],
        out_specs=pl.BlockSpec(memory_space=pltpu.SMEM),
        out_shape=jax.ShapeDtypeStruct((1,), jnp.float32),
        scratch_shapes=[
            pltpu.SMEM((4,), jnp.float32),
            pltpu.VMEM((1, 512), jnp.float32),
            pltpu.VMEM((_B, 1, 512, 512), jnp.bfloat16),
        ],
        compiler_params=pltpu.CompilerParams(
            dimension_semantics=("arbitrary", "arbitrary"),
            vmem_limit_bytes=64 * 1024 * 1024,
        ),
    )(prediction, target)
    return out.reshape(())


# speedup vs baseline: 1.5274x; 1.0074x over previous
import jax
import jax.numpy as jnp
from jax.experimental import pallas as pl
from jax.experimental.pallas import tpu as pltpu

_B = 64
_BB = 4
_NBLK = _B // _BB


def _berhu_body(x_ref, y_ref, out_ref, acc_ref, vacc_ref, cache_ref):
    p = pl.program_id(0)
    j = pl.program_id(1)

    @pl.when(p == 0)
    def _():
        d = jnp.abs(x_ref[...] - y_ref[...])

        @pl.when(j == 0)
        def _():
            acc_ref[0] = 0.0
            acc_ref[1] = 0.0

        acc_ref[0] = jnp.maximum(acc_ref[0], jnp.max(d))
        acc_ref[1] += jnp.sum(d)
        cache_ref[pl.ds(j * _BB, _BB)] = d.astype(jnp.bfloat16)

    @pl.when(jnp.logical_and(p == 1, j < _NBLK // 4))
    def _():
        @pl.when(j == 0)
        def _():
            vacc_ref[...] = jnp.zeros_like(vacc_ref)

        cb = jnp.maximum(acc_ref[0] * 0.2, 0.0001).astype(jnp.bfloat16)
        d = cache_ref[pl.ds(j * 4 * _BB, 4 * _BB)].reshape(4 * _BB * 512, 512)
        u = jnp.maximum(d - cb, jnp.bfloat16(0.0))
        ones = jnp.ones((4 * _BB * 512,), jnp.bfloat16)
        sq = jax.lax.dot_general(ones, u * u, (((0,), (0,)), ((), ())),
                                 preferred_element_type=jnp.float32)
        vacc_ref[0, :] += sq

    @pl.when(jnp.logical_and(p == 1, j == _NBLK - 1))
    def _():
        cb = jnp.maximum(acc_ref[0] * 0.2, 0.0001).astype(jnp.bfloat16)
        c32 = cb.astype(jnp.float32)
        out_ref[0] = acc_ref[1] + jnp.sum(vacc_ref[0, :]) / (2.0 * c32)


def kernel(prediction, target):
    spec = pl.BlockSpec(
        (_BB, 1, 512, 512), lambda p, j: (jnp.where(p == 0, j, 0), 0, 0, 0))
    out = pl.pallas_call(
        _berhu_body,
        grid=(2, _NBLK),
        in_specs=[spec, spec],
        out_specs=pl.BlockSpec(memory_space=pltpu.SMEM),
        out_shape=jax.ShapeDtypeStruct((1,), jnp.float32),
        scratch_shapes=[
            pltpu.SMEM((4,), jnp.float32),
            pltpu.VMEM((1, 512), jnp.float32),
            pltpu.VMEM((_B, 1, 512, 512), jnp.bfloat16),
        ],
        compiler_params=pltpu.CompilerParams(
            dimension_semantics=("arbitrary", "arbitrary"),
            vmem_limit_bytes=64 * 1024 * 1024,
        ),
    )(prediction, target)
    return out.reshape(())
